# Initial kernel scaffold; baseline (speedup 1.0000x reference)
import jax
import jax.numpy as jnp
from jax.experimental import pallas as pl


def kernel(seed_feats, mask_weight, unmasked_idx, masked_idx):
    raise NotImplementedError("write your pallas kernel here")


# speedup vs baseline: 21.5701x; 21.5701x over previous
import jax
import jax.numpy as jnp
from jax import lax
from jax.experimental import pallas as pl
from jax.experimental.pallas import tpu as pltpu, tpu_sc as plsc

BEV_M = 200 * 200 * 16
NC, NS = 2, 16
NW = NC * NS
CHUNK = 128


def _make_kernel(NU, NM, D):
    per_u = NU // NW
    per_m = NM // NW
    full_u, tail_u = per_u // CHUNK, per_u % CHUNK
    full_m, tail_m = per_m // CHUNK, per_m % CHUNK
    assert NU % NW == 0 and NM % NW == 0
    assert tail_u % 8 == 0 and tail_m % 8 == 0

    mesh = plsc.VectorSubcoreMesh(
        core_axis_name="c", subcore_axis_name="s", num_cores=NC, num_subcores=NS
    )

    @jax.jit
    def run(seed_feats, mask_rows, uidx, midx):
        @pl.kernel(
            out_type=jax.ShapeDtypeStruct((BEV_M, D), jnp.float32),
            mesh=mesh,
            scratch_types=[
                pltpu.VMEM((CHUNK,), jnp.int32),
                pltpu.VMEM((CHUNK,), jnp.int32),
                pltpu.VMEM((tail_u,), jnp.int32),
                pltpu.VMEM((CHUNK, D), jnp.float32),
                pltpu.VMEM((CHUNK, D), jnp.float32),
                pltpu.SemaphoreType.DMA,
                pltpu.SemaphoreType.DMA,
                pltpu.SemaphoreType.DMA,
            ],
        )
        def k(seed_hbm, mrows_hbm, uidx_hbm, midx_hbm, out_hbm,
              idx_a, idx_b, idx_t, rows_v, mrows_v, sem_i, sem_r, sem_s):
            wid = lax.axis_index("s") * NC + lax.axis_index("c")
            ubase = wid * per_u
            mbase = wid * per_m

            pltpu.sync_copy(mrows_hbm, mrows_v)

            def u_body(j, _):
                off = ubase + j * CHUNK
                ci = pltpu.async_copy(uidx_hbm.at[pl.ds(off, CHUNK)], idx_a, sem_i)
                cr = pltpu.async_copy(seed_hbm.at[pl.ds(off, CHUNK)], rows_v, sem_r)
                ci.wait()
                cr.wait()
                pltpu.async_copy(rows_v, out_hbm.at[idx_a], sem_s).wait()
                return 0

            lax.fori_loop(0, full_u, u_body, 0)

            if tail_u:
                off = ubase + full_u * CHUNK
                ci = pltpu.async_copy(uidx_hbm.at[pl.ds(off, tail_u)], idx_t, sem_i)
                cr = pltpu.async_copy(
                    seed_hbm.at[pl.ds(off, tail_u)],
                    rows_v.at[pl.ds(0, tail_u)], sem_r)
                ci.wait()
                cr.wait()
                pltpu.async_copy(
                    rows_v.at[pl.ds(0, tail_u)], out_hbm.at[idx_t], sem_s).wait()

            def m_body(j, _):
                off = mbase + j * CHUNK
                pltpu.async_copy(midx_hbm.at[pl.ds(off, CHUNK)], idx_b, sem_i).wait()
                pltpu.async_copy(mrows_v, out_hbm.at[idx_b], sem_s).wait()
                return 0

            lax.fori_loop(0, full_m, m_body, 0)

            if tail_m:
                off = mbase + full_m * CHUNK
                pltpu.async_copy(midx_hbm.at[pl.ds(off, tail_m)], idx_t, sem_i).wait()
                pltpu.async_copy(
                    mrows_v.at[pl.ds(0, tail_m)], out_hbm.at[idx_t], sem_s).wait()

        return k(seed_feats, mask_rows, uidx, midx)

    return run


def kernel(seed_feats, mask_weight, unmasked_idx, masked_idx):
    NU, D = seed_feats.shape
    NM = masked_idx.shape[0]
    run = _make_kernel(NU, NM, D)
    mask_rows = jnp.broadcast_to(
        mask_weight.reshape(1, D).astype(jnp.float32), (CHUNK, D))
    return run(
        seed_feats,
        mask_rows,
        unmasked_idx.astype(jnp.int32),
        masked_idx.astype(jnp.int32),
    )
